# Initial kernel scaffold; baseline (speedup 1.0000x reference)
import jax
import jax.numpy as jnp
from jax.experimental import pallas as pl


def kernel(x_cfg, x_feat, x_op, edge_index, emb, W1, b1, W2, b2, W3, b3, D1w, D1b, D2w, D2b, D3w, D3b):
    raise NotImplementedError("write your pallas kernel here")


# speedup vs baseline: 7.2243x; 7.2243x over previous
import functools

import jax
import jax.numpy as jnp
from jax import lax
from jax.experimental import pallas as pl
from jax.experimental.pallas import tpu as pltpu
from jax.experimental.pallas import tpu_sc as plsc

N = 10000
E = 320000
H = 512
NG = 4
GW = H // NG
NC = 2
NS = 16
EROWS = E // 128
RPT = 624
RPT_LAST = N - 15 * RPT
BM = 1000
NB = N // BM

_MESH = plsc.VectorSubcoreMesh(
    core_axis_name="c", subcore_axis_name="s", num_cores=NC, num_subcores=NS)


def _copy_own(sid, copy_fn):
    @pl.when(sid < NS - 1)
    def _():
        copy_fn(sid * RPT, RPT)

    @pl.when(sid == NS - 1)
    def _():
        copy_fn((NS - 1) * RPT, RPT_LAST)


def _deg_body(dst_hbm, ones_hbm, zeros_hbm, out_hbm, didx, ones_v, hist_sh):
    cid = lax.axis_index("c")
    sid = lax.axis_index("s")
    pltpu.sync_copy(ones_hbm, ones_v)
    _copy_own(sid, lambda s, n: pltpu.sync_copy(
        zeros_hbm.at[pl.ds(0, n)], hist_sh.at[pl.ds(s, n)]))
    plsc.subcore_barrier()
    base = cid * (EROWS // NC)
    cnt = 78 + jnp.where(sid < 2, 1, 0)

    def body(i, _):
        r = base + sid + NS * i
        pltpu.sync_copy(dst_hbm.at[r], didx)
        pltpu.sync_copy(ones_v, hist_sh.at[didx], add=True)
        return 0

    lax.fori_loop(0, cnt, body, 0)
    plsc.subcore_barrier()
    _copy_own(sid, lambda s, n: pltpu.sync_copy(
        hist_sh.at[pl.ds(s, n)], out_hbm.at[cid].at[pl.ds(s, n)]))


def _deg_call(dst2d, ones_col, zeros_col):
    return pl.kernel(
        _deg_body,
        out_type=jax.ShapeDtypeStruct((NC, N, 1), jnp.float32),
        mesh=_MESH,
        scratch_types=[
            pltpu.VMEM((128,), jnp.int32),
            pltpu.VMEM((128, 1), jnp.float32),
            pltpu.VMEM_SHARED((N, 1), jnp.float32),
        ],
    )(dst2d, ones_col, zeros_col)


def _agg_body(hs_hbm, src_hbm, dst_hbm, zeros_hbm, out_hbm,
              sidx, didx, rows_v, acc_sh, sem):
    cid = lax.axis_index("c")
    sid = lax.axis_index("s")

    def zero_own():
        _copy_own(sid, lambda s, n: pltpu.sync_copy(
            zeros_hbm.at[pl.ds(0, n)], acc_sh.at[pl.ds(s, n)]))

    zero_own()
    plsc.subcore_barrier()
    cnt = 156 + jnp.where(sid < 4, 1, 0)

    for p in range(NG // NC):
        g = NC * cid + p

        def body(i, _):
            r = sid + NS * i
            pltpu.sync_copy(src_hbm.at[r], sidx)
            pltpu.sync_copy(dst_hbm.at[r], didx)
            pltpu.async_copy(hs_hbm.at[g].at[sidx], rows_v, sem).wait()
            pltpu.sync_copy(rows_v, acc_sh.at[didx], add=True)
            return 0

        lax.fori_loop(0, cnt, body, 0)
        plsc.subcore_barrier()
        _copy_own(sid, lambda s, n: pltpu.sync_copy(
            acc_sh.at[pl.ds(s, n)], out_hbm.at[g].at[pl.ds(s, n)]))
        if p + 1 < NG // NC:
            zero_own()
        plsc.subcore_barrier()


def _agg_call(hs, src2d, dst2d, zeros_blk):
    return pl.kernel(
        _agg_body,
        out_type=jax.ShapeDtypeStruct((NG, N, GW), jnp.float32),
        mesh=_MESH,
        scratch_types=[
            pltpu.VMEM((128,), jnp.int32),
            pltpu.VMEM((128,), jnp.int32),
            pltpu.VMEM((128, GW), jnp.float32),
            pltpu.VMEM_SHARED((N, GW), jnp.float32),
            pltpu.SemaphoreType.DMA,
        ],
    )(hs, src2d, dst2d, zeros_blk)


def _tc1_body(hist_ref, xf_ref, xop_ref, emb_ref, w1_ref, hs_ref, dis_ref):
    deg = hist_ref[0] + hist_ref[1] + 1.0
    dis = lax.rsqrt(deg)
    dis_ref[...] = dis
    oh = (xop_ref[...] == lax.broadcasted_iota(jnp.int32, (BM, 120), 1))
    xe = jnp.dot(oh.astype(jnp.float32), emb_ref[...],
                 preferred_element_type=jnp.float32)
    x = jnp.concatenate([xf_ref[...], xe], axis=1)
    hs = jnp.dot(x, w1_ref[...], preferred_element_type=jnp.float32) * dis
    for g in range(NG):
        hs_ref[g] = hs[:, g * GW:(g + 1) * GW]


def _tc1_call(deg2, x_feat, x_op2, emb, W1):
    return pl.pallas_call(
        _tc1_body,
        grid=(NB,),
        in_specs=[
            pl.BlockSpec((NC, BM, 1), lambda i: (0, i, 0)),
            pl.BlockSpec((BM, 140), lambda i: (i, 0)),
            pl.BlockSpec((BM, 1), lambda i: (i, 0)),
            pl.BlockSpec((120, 4), lambda i: (0, 0)),
            pl.BlockSpec((144, H), lambda i: (0, 0)),
        ],
        out_specs=[
            pl.BlockSpec((NG, BM, GW), lambda i: (0, i, 0)),
            pl.BlockSpec((BM, 1), lambda i: (i, 0)),
        ],
        out_shape=[
            jax.ShapeDtypeStruct((NG, N, GW), jnp.float32),
            jax.ShapeDtypeStruct((N, 1), jnp.float32),
        ],
    )(deg2, x_feat, x_op2, emb, W1)


def _mid_body(agg_ref, hs_ref, dis_ref, b_ref, w_ref, o_ref):
    dis = dis_ref[...]
    x = jnp.concatenate(
        [agg_ref[g] + hs_ref[g] for g in range(NG)], axis=1)
    x = jnp.maximum(x * dis + b_ref[...], 0.0)
    hs = jnp.dot(x, w_ref[...], preferred_element_type=jnp.float32) * dis
    for g in range(NG):
        o_ref[g] = hs[:, g * GW:(g + 1) * GW]


def _mid_call(agg, hs, dis2, bvec, W):
    return pl.pallas_call(
        _mid_body,
        grid=(NB,),
        in_specs=[
            pl.BlockSpec((NG, BM, GW), lambda i: (0, i, 0)),
            pl.BlockSpec((NG, BM, GW), lambda i: (0, i, 0)),
            pl.BlockSpec((BM, 1), lambda i: (i, 0)),
            pl.BlockSpec((1, H), lambda i: (0, 0)),
            pl.BlockSpec((H, H), lambda i: (0, 0)),
        ],
        out_specs=pl.BlockSpec((NG, BM, GW), lambda i: (0, i, 0)),
        out_shape=jax.ShapeDtypeStruct((NG, N, GW), jnp.float32),
    )(agg, hs, dis2, bvec, W)


def _fin_body(agg_ref, hs_ref, dis_ref, b_ref, xcfg_ref, d1w_ref, d1b_ref,
              d2w_ref, d2b_ref, d3w_ref, d3b_ref, o_ref, acc_ref):
    i = pl.program_id(0)

    @pl.when(i == 0)
    def _():
        acc_ref[...] = jnp.zeros_like(acc_ref)

    x = jnp.concatenate(
        [agg_ref[g] + hs_ref[g] for g in range(NG)], axis=1)
    x = jnp.maximum(x * dis_ref[...] + b_ref[...], 0.0)
    acc_ref[...] += jnp.sum(x, axis=0, keepdims=True)

    @pl.when(i == NB - 1)
    def _():
        pooled = acc_ref[...] * (1.0 / N)
        z = jnp.dot(xcfg_ref[...], d1w_ref[0:24, :],
                    preferred_element_type=jnp.float32)
        z += jnp.dot(pooled, d1w_ref[24:24 + H, :],
                     preferred_element_type=jnp.float32)
        z = jnp.maximum(z + d1b_ref[...], 0.0)
        z = jnp.maximum(
            jnp.dot(z, d2w_ref[...], preferred_element_type=jnp.float32)
            + d2b_ref[...], 0.0)
        z = (jnp.dot(z, d3w_ref[...], preferred_element_type=jnp.float32)
             + d3b_ref[...])
        m = jnp.mean(z)
        c = z - m
        v = jnp.sum(c * c) * (1.0 / (z.shape[0] - 1))
        o_ref[...] = c / (jnp.sqrt(v) + 1e-5)


def _fin_call(agg, hs, dis2, bvec, x_cfg, D1w, D1b, D2w, D2b, D3w, D3b):
    ncfg = x_cfg.shape[0]
    return pl.pallas_call(
        _fin_body,
        grid=(NB,),
        in_specs=[
            pl.BlockSpec((NG, BM, GW), lambda i: (0, i, 0)),
            pl.BlockSpec((NG, BM, GW), lambda i: (0, i, 0)),
            pl.BlockSpec((BM, 1), lambda i: (i, 0)),
            pl.BlockSpec((1, H), lambda i: (0, 0)),
            pl.BlockSpec((ncfg, 24), lambda i: (0, 0)),
            pl.BlockSpec((24 + H, 64), lambda i: (0, 0)),
            pl.BlockSpec((1, 64), lambda i: (0, 0)),
            pl.BlockSpec((64, 64), lambda i: (0, 0)),
            pl.BlockSpec((1, 64), lambda i: (0, 0)),
            pl.BlockSpec((64, 1), lambda i: (0, 0)),
            pl.BlockSpec((1, 1), lambda i: (0, 0)),
        ],
        out_specs=pl.BlockSpec((ncfg, 1), lambda i: (0, 0)),
        out_shape=jax.ShapeDtypeStruct((ncfg, 1), jnp.float32),
        scratch_shapes=[pltpu.VMEM((1, H), jnp.float32)],
    )(agg, hs, dis2, bvec, x_cfg, D1w, D1b, D2w, D2b, D3w, D3b)


def kernel(x_cfg, x_feat, x_op, edge_index, emb, W1, b1, W2, b2, W3, b3,
           D1w, D1b, D2w, D2b, D3w, D3b):
    src2d = edge_index[0].reshape(EROWS, 128)
    dst2d = edge_index[1].reshape(EROWS, 128)
    x_op2 = x_op.reshape(N, 1)
    ones_col = jnp.ones((128, 1), jnp.float32)
    zeros_col = jnp.zeros((RPT_LAST, 1), jnp.float32)
    zeros_blk = jnp.zeros((RPT_LAST, GW), jnp.float32)

    deg2 = _deg_call(dst2d, ones_col, zeros_col)
    hs1, dis2 = _tc1_call(deg2, x_feat, x_op2, emb, W1)
    agg1 = _agg_call(hs1, src2d, dst2d, zeros_blk)
    hs2 = _mid_call(agg1, hs1, dis2, b1.reshape(1, H), W2)
    agg2 = _agg_call(hs2, src2d, dst2d, zeros_blk)
    hs3 = _mid_call(agg2, hs2, dis2, b2.reshape(1, H), W3)
    agg3 = _agg_call(hs3, src2d, dst2d, zeros_blk)
    z = _fin_call(agg3, hs3, dis2, b3.reshape(1, H), x_cfg,
                  D1w, D1b.reshape(1, 64), D2w, D2b.reshape(1, 64),
                  D3w, D3b.reshape(1, 1))
    return z.reshape(-1)
